# baseline (speedup 1.0000x reference)
import functools
import math

import jax
import jax.numpy as jnp
from jax import lax
from jax.experimental import pallas as pl
from jax.experimental.pallas import tpu as pltpu
from jax.experimental.pallas import tpu_sc as plsc

HID = 128
LANES = 16
VPR = HID // LANES

_info = plsc.get_sparse_core_info()
NC, NS = _info.num_cores, _info.num_subcores
NW = NC * NS

SCALE = math.sqrt(float(HID))
NSLOT = 2


def _make_kernel(b: int, s: int):
    assert s % NW == 0
    spw = s // NW
    ch = min(64, spw)
    n_sub = spw // ch

    mesh = plsc.VectorSubcoreMesh(core_axis_name="c", subcore_axis_name="s")

    @functools.partial(
        pl.kernel,
        out_type=jax.ShapeDtypeStruct((b, s, HID), jnp.float32),
        mesh=mesh,
        scratch_types=[
            pltpu.VMEM((b, spw), jnp.int32),
            pltpu.VMEM((spw, HID), jnp.float32),
            [[pltpu.VMEM((ch, HID), jnp.float32)] * b] * NSLOT,
            [[pltpu.SemaphoreType.DMA] * b] * NSLOT,
            [[pltpu.SemaphoreType.DMA] * b] * NSLOT,
            pltpu.SemaphoreType.DMA,
            pltpu.SemaphoreType.DMA,
        ],
    )
    def body(tok_hbm, idx_hbm, pos_hbm, out_hbm, idx_v, pos_v, bufs,
             gsems, osems, isem, psem):
        wid = lax.axis_index("s") * NC + lax.axis_index("c")
        s_base = wid * spw

        icp = pltpu.async_copy(idx_hbm.at[:, pl.ds(s_base, spw)], idx_v, isem)
        pcp = pltpu.async_copy(pos_hbm.at[pl.ds(s_base, spw)], pos_v, psem)

        scale = jnp.full((LANES,), SCALE, dtype=jnp.float32)

        def start_gathers(h):
            sl = h % NSLOT
            return [
                pltpu.async_copy(
                    tok_hbm.at[idx_v.at[bb, pl.ds(h * ch, ch)]],
                    bufs[sl][bb], gsems[sl][bb])
                for bb in range(b)
            ]

        depth = min(NSLOT, n_sub)
        icp.wait()
        copies = {h: start_gathers(h) for h in range(depth)}
        out_copies = {}
        pcp.wait()
        for h in range(n_sub):
            sl = h % NSLOT
            for c in copies[h]:
                c.wait()
            slot = bufs[sl]
            pbase = h * ch

            @plsc.parallel_loop(0, ch, unroll=4)
            def row(r):
                for j in range(VPR):
                    vsl = pl.ds(j * LANES, LANES)
                    pv = pos_v[pbase + r, vsl]
                    for bb in range(b):
                        slot[bb][r, vsl] = slot[bb][r, vsl] * scale + pv

            out_copies[h] = [
                pltpu.async_copy(
                    slot[bb], out_hbm.at[bb, pl.ds(s_base + h * ch, ch)],
                    osems[sl][bb])
                for bb in range(b)
            ]
            if h + depth < n_sub:
                prev = h + depth - NSLOT
                if prev >= 0:
                    for c in out_copies[prev]:
                        c.wait()
                copies[h + depth] = start_gathers(h + depth)
        for h in range(max(0, n_sub - NSLOT), n_sub):
            for c in out_copies[h]:
                c.wait()

    return body


@jax.jit
def kernel(x, token_table, pos_table):
    b, s = x.shape
    out = _make_kernel(b, s)(token_table, x.astype(jnp.int32), pos_table)
    return out
